# baseline (speedup 1.0000x reference)
import functools

import jax
import jax.numpy as jnp
from jax import lax
from jax.experimental import pallas as pl
from jax.experimental.pallas import tpu as pltpu
from jax.experimental.pallas import tpu_sc as plsc

L = 16
K = 64
NC, NS = 2, 16
NW = NC * NS
G = 32


def _sa(v):
    return lax.sort(v)


def _sd(v):
    return plsc.sort_key_val(v, v, descending=True)[0]


def _sort_dir(v, asc):
    return _sa(v) if asc else _sd(v)


def _tree(leaves, top):
    nodes = leaves
    while len(nodes) > 1:
        n = len(nodes) // 2
        new = []
        for i in range(n):
            m = (jnp.maximum if top else jnp.minimum)(
                nodes[2 * i], nodes[2 * i + 1])
            asc = False if n == 1 else (i % 2 == 0)
            new.append(_sort_dir(m, asc))
        nodes = new
    return nodes[0]


def _rev(v):
    return lax.rev(v, (0,))


def _merge16(a_asc, b_desc):
    return _sa(jnp.minimum(a_asc, b_desc)), _sa(jnp.maximum(a_asc, b_desc))


def _merge32(a, b):
    rb0, rb1 = _rev(b[1]), _rev(b[0])
    l0, h0 = jnp.minimum(a[0], rb0), jnp.maximum(a[0], rb0)
    l1, h1 = jnp.minimum(a[1], rb1), jnp.maximum(a[1], rb1)
    return (_sa(jnp.minimum(l0, l1)), _sa(jnp.maximum(l0, l1)),
            _sa(jnp.minimum(h0, h1)), _sa(jnp.maximum(h0, h1)))


def _bitonic64(h):
    p02, q02 = jnp.minimum(h[0], h[2]), jnp.maximum(h[0], h[2])
    p13, q13 = jnp.minimum(h[1], h[3]), jnp.maximum(h[1], h[3])
    return (_sa(jnp.minimum(p02, p13)), _sa(jnp.maximum(p02, p13)),
            _sa(jnp.minimum(q02, q13)), _sa(jnp.maximum(q02, q13)))


def _merge_top64(a, b):
    return _bitonic64([jnp.maximum(a[i], _rev(b[3 - i])) for i in range(4)])


def _merge_bot64(a, b):
    return _bitonic64([jnp.minimum(a[i], _rev(b[3 - i])) for i in range(4)])


def _block64(leaves, top):
    s32 = [_merge16(leaves[2 * i], leaves[2 * i + 1]) for i in range(16)]
    s64 = [_merge32(s32[2 * i], s32[2 * i + 1]) for i in range(8)]
    f = _merge_top64 if top else _merge_bot64
    while len(s64) > 1:
        s64 = [f(s64[2 * i], s64[2 * i + 1]) for i in range(len(s64) // 2)]
    return s64[0]


def _make_kernel(rows, n):
    ngrp = n // (L * G)
    rows_per = rows // NW
    mesh = plsc.VectorSubcoreMesh(core_axis_name="c", subcore_axis_name="s")

    @functools.partial(
        pl.kernel,
        mesh=mesh,
        out_type=jax.ShapeDtypeStruct((rows, 2 * K), jnp.float32),
        scratch_types=[
            pltpu.VMEM((n,), jnp.float32),
            pltpu.VMEM((2 * K,), jnp.float32),
        ],
        compiler_params=pltpu.CompilerParams(needs_layout_passes=False),
    )
    def k(x_hbm, out_hbm, data_v, out_v):
        wid = lax.axis_index("s") * NC + lax.axis_index("c")

        def row_body(r, carry_none):
            row = wid * rows_per + r
            pltpu.sync_copy(x_hbm.at[row, 0], data_v)

            nv = jnp.full((L,), -jnp.inf, jnp.float32)
            pv = jnp.full((L,), jnp.inf, jnp.float32)
            init = (nv, nv, nv, nv, pv, pv, pv, pv)

            def grp_body(g, carry):
                t0, t1, t2, t3, b0, b1, b2, b3 = carry
                base = g * (L * G)
                raw = [data_v[pl.ds(base + j * L, L)] for j in range(G)]
                leaves = [_sort_dir(raw[j], j % 2 == 0) for j in range(G)]
                hi = _tree(leaves, True)
                lo = _tree(leaves, False)
                thr_t = t0[0]
                thr_b = b3[L - 1]

                def top_fb(c):
                    lv = [_sort_dir(data_v[pl.ds(base + j * L, L)],
                                    j % 2 == 0) for j in range(G)]
                    return _merge_top64(c, _block64(lv, True))

                def top_ok(c):
                    return _bitonic64(
                        [jnp.maximum(c[0], hi), c[1], c[2], c[3]])

                t0, t1, t2, t3 = lax.cond(
                    hi[L - 1] > thr_t, top_fb, top_ok, (t0, t1, t2, t3))

                def bot_fb(c):
                    lv = [_sort_dir(data_v[pl.ds(base + j * L, L)],
                                    j % 2 == 0) for j in range(G)]
                    return _merge_bot64(c, _block64(lv, False))

                def bot_ok(c):
                    return _bitonic64(
                        [c[0], c[1], c[2], jnp.minimum(c[3], lo)])

                b0, b1, b2, b3 = lax.cond(
                    lo[0] < thr_b, bot_fb, bot_ok, (b0, b1, b2, b3))
                return (t0, t1, t2, t3, b0, b1, b2, b3)

            t0, t1, t2, t3, b0, b1, b2, b3 = lax.fori_loop(
                0, ngrp, grp_body, init, unroll=4)

            out_v[pl.ds(0 * L, L)] = lax.rev(t3, (0,))
            out_v[pl.ds(1 * L, L)] = lax.rev(t2, (0,))
            out_v[pl.ds(2 * L, L)] = lax.rev(t1, (0,))
            out_v[pl.ds(3 * L, L)] = lax.rev(t0, (0,))
            out_v[pl.ds(4 * L, L)] = lax.rev(b3, (0,))
            out_v[pl.ds(5 * L, L)] = lax.rev(b2, (0,))
            out_v[pl.ds(6 * L, L)] = lax.rev(b1, (0,))
            out_v[pl.ds(7 * L, L)] = lax.rev(b0, (0,))
            pltpu.sync_copy(out_v, out_hbm.at[row])
            return carry_none

        lax.fori_loop(0, rows_per, row_body, 0)

    return k


@jax.jit
def kernel(x):
    rows = x.shape[0]
    n = x.shape[2]
    return _make_kernel(rows, n)(x)


# speedup vs baseline: 2.0050x; 2.0050x over previous
import functools

import jax
import jax.numpy as jnp
from jax import lax
from jax.experimental import pallas as pl
from jax.experimental.pallas import tpu as pltpu
from jax.experimental.pallas import tpu_sc as plsc

L = 16
K = 64
NC, NS = 2, 16
NW = NC * NS
G = 32


def _sa(v):
    return lax.sort(v)


def _sd(v):
    return plsc.sort_key_val(v, v, descending=True)[0]


def _sort_dir(v, asc):
    return _sa(v) if asc else _sd(v)


def _tree(leaves, top):
    nodes = leaves
    while len(nodes) > 1:
        n = len(nodes) // 2
        new = []
        for i in range(n):
            m = (jnp.maximum if top else jnp.minimum)(
                nodes[2 * i], nodes[2 * i + 1])
            asc = False if n == 1 else (i % 2 == 0)
            new.append(_sort_dir(m, asc))
        nodes = new
    return nodes[0]


def _rev(v):
    return lax.rev(v, (0,))


def _merge16(a_asc, b_desc):
    return _sa(jnp.minimum(a_asc, b_desc)), _sa(jnp.maximum(a_asc, b_desc))


def _merge32(a, b):
    rb0, rb1 = _rev(b[1]), _rev(b[0])
    l0, h0 = jnp.minimum(a[0], rb0), jnp.maximum(a[0], rb0)
    l1, h1 = jnp.minimum(a[1], rb1), jnp.maximum(a[1], rb1)
    return (_sa(jnp.minimum(l0, l1)), _sa(jnp.maximum(l0, l1)),
            _sa(jnp.minimum(h0, h1)), _sa(jnp.maximum(h0, h1)))


def _bitonic64(h):
    p02, q02 = jnp.minimum(h[0], h[2]), jnp.maximum(h[0], h[2])
    p13, q13 = jnp.minimum(h[1], h[3]), jnp.maximum(h[1], h[3])
    return (_sa(jnp.minimum(p02, p13)), _sa(jnp.maximum(p02, p13)),
            _sa(jnp.minimum(q02, q13)), _sa(jnp.maximum(q02, q13)))


def _merge_top64(a, b):
    return _bitonic64([jnp.maximum(a[i], _rev(b[3 - i])) for i in range(4)])


def _merge_bot64(a, b):
    return _bitonic64([jnp.minimum(a[i], _rev(b[3 - i])) for i in range(4)])


def _block64(leaves, top):
    s32 = [_merge16(leaves[2 * i], leaves[2 * i + 1]) for i in range(16)]
    s64 = [_merge32(s32[2 * i], s32[2 * i + 1]) for i in range(8)]
    f = _merge_top64 if top else _merge_bot64
    while len(s64) > 1:
        s64 = [f(s64[2 * i], s64[2 * i + 1]) for i in range(len(s64) // 2)]
    return s64[0]


def _make_kernel(rows, n):
    ngrp = n // (L * G)
    rows_per = rows // NW
    mesh = plsc.VectorSubcoreMesh(core_axis_name="c", subcore_axis_name="s")

    @functools.partial(
        pl.kernel,
        mesh=mesh,
        out_type=jax.ShapeDtypeStruct((rows, 2 * K), jnp.float32),
        scratch_types=[
            pltpu.VMEM((2, n), jnp.float32),
            pltpu.VMEM((2 * K,), jnp.float32),
            pltpu.SemaphoreType.DMA((2,)),
        ],
        compiler_params=pltpu.CompilerParams(needs_layout_passes=False),
    )
    def k(x_hbm, out_hbm, data2_v, out_v, sem):
        wid = lax.axis_index("s") * NC + lax.axis_index("c")
        row0 = wid * rows_per
        pltpu.async_copy(x_hbm.at[row0], data2_v.at[pl.ds(0, 1)], sem.at[0])

        def row_body(r, carry_none):
            row = row0 + r
            buf = lax.rem(r, 2)
            pltpu.make_async_copy(
                x_hbm.at[row], data2_v.at[pl.ds(buf, 1)], sem.at[buf]).wait()

            @pl.when(r + 1 < rows_per)
            def _():
                pltpu.async_copy(x_hbm.at[row + 1],
                                 data2_v.at[pl.ds(1 - buf, 1)],
                                 sem.at[1 - buf])

            nv = jnp.full((L,), -jnp.inf, jnp.float32)
            pv = jnp.full((L,), jnp.inf, jnp.float32)
            init = (nv, nv, nv, nv, pv, pv, pv, pv)

            def grp_body(g, carry):
                t0, t1, t2, t3, b0, b1, b2, b3 = carry
                base = g * (L * G)
                raw = [data2_v[buf, pl.ds(base + j * L, L)]
                       for j in range(G)]
                leaves = [_sort_dir(raw[j], j % 2 == 0) for j in range(G)]
                hi = _tree(leaves, True)
                lo = _tree(leaves, False)
                thr_t = t0[0]
                thr_b = b3[L - 1]

                def top_fb(c):
                    lv = [_sort_dir(data2_v[buf, pl.ds(base + j * L, L)],
                                    j % 2 == 0) for j in range(G)]
                    return _merge_top64(c, _block64(lv, True))

                def top_ok(c):
                    return _bitonic64(
                        [jnp.maximum(c[0], hi), c[1], c[2], c[3]])

                t0, t1, t2, t3 = lax.cond(
                    hi[L - 1] > thr_t, top_fb, top_ok, (t0, t1, t2, t3))

                def bot_fb(c):
                    lv = [_sort_dir(data2_v[buf, pl.ds(base + j * L, L)],
                                    j % 2 == 0) for j in range(G)]
                    return _merge_bot64(c, _block64(lv, False))

                def bot_ok(c):
                    return _bitonic64(
                        [c[0], c[1], c[2], jnp.minimum(c[3], lo)])

                b0, b1, b2, b3 = lax.cond(
                    lo[0] < thr_b, bot_fb, bot_ok, (b0, b1, b2, b3))
                return (t0, t1, t2, t3, b0, b1, b2, b3)

            t0, t1, t2, t3, b0, b1, b2, b3 = lax.fori_loop(
                0, ngrp, grp_body, init, unroll=2)

            out_v[pl.ds(0 * L, L)] = lax.rev(t3, (0,))
            out_v[pl.ds(1 * L, L)] = lax.rev(t2, (0,))
            out_v[pl.ds(2 * L, L)] = lax.rev(t1, (0,))
            out_v[pl.ds(3 * L, L)] = lax.rev(t0, (0,))
            out_v[pl.ds(4 * L, L)] = lax.rev(b3, (0,))
            out_v[pl.ds(5 * L, L)] = lax.rev(b2, (0,))
            out_v[pl.ds(6 * L, L)] = lax.rev(b1, (0,))
            out_v[pl.ds(7 * L, L)] = lax.rev(b0, (0,))
            pltpu.sync_copy(out_v, out_hbm.at[row])
            return carry_none

        lax.fori_loop(0, rows_per, row_body, 0)

    return k


@jax.jit
def kernel(x):
    rows = x.shape[0]
    n = x.shape[2]
    return _make_kernel(rows, n)(x)
